# baseline (speedup 1.0000x reference)
import functools
import math

import jax
import jax.numpy as jnp
from jax import lax
from jax.experimental import pallas as pl
from jax.experimental.pallas import tpu as pltpu
from jax.experimental.pallas import tpu_sc as plsc

_D = 2048
_H = 16
_DH = 128
_E = 8
_TOPK = 2
_DFF = 4096
_DP = 256
_NPRIM = 128
_T = 2048
_ROPE_THETA = 10000.0
_EPS = 1e-6

_BT = 256
_BN = 128
_NPAD = 4096 + _E * _BN
_NM = _NPAD // _BN
_BF = 2048
_NF = _DFF // _BF


def _rms(x, w):
    var = jnp.mean(x * x, axis=-1, keepdims=True)
    return x * lax.rsqrt(var + _EPS) * w


def _qkv_body(x_ref, w_ref, n1_ref, q_ref):
    xn = _rms(x_ref[...], n1_ref[...])
    w = w_ref[0]
    q_ref[0] = lax.dot_general(xn, w, (((1,), (1,)), ((), ())),
                               preferred_element_type=jnp.float32)


def _qkv_call(x2d, qkv_w3, n1w):
    nt = _T // _BT
    return pl.pallas_call(
        _qkv_body,
        grid=(3, nt),
        in_specs=[
            pl.BlockSpec((_BT, _D), lambda j, t: (t, 0)),
            pl.BlockSpec((1, _D, _D), lambda j, t: (j, 0, 0)),
            pl.BlockSpec((1, _D), lambda j, t: (0, 0)),
        ],
        out_specs=pl.BlockSpec((1, _BT, _D), lambda j, t: (j, t, 0)),
        out_shape=jax.ShapeDtypeStruct((3, _T, _D), jnp.float32),
    )(x2d, qkv_w3, n1w)


def _rot_half(u):
    u1 = u[:, : _DH // 2]
    u2 = u[:, _DH // 2:]
    return jnp.concatenate([-u2, u1], axis=1)


def _attn_body(q_ref, k_ref, v_ref, cq_ref, sq_ref, ck_ref, sk_ref, o_ref):
    iq = pl.program_id(1)
    q = q_ref[0]
    k = k_ref[0]
    v = v_ref[0]
    qr = q * cq_ref[...] + _rot_half(q) * sq_ref[...]
    kr = k * ck_ref[...] + _rot_half(k) * sk_ref[...]
    s = lax.dot_general(qr, kr, (((1,), (1,)), ((), ())),
                        preferred_element_type=jnp.float32)
    s = s * (1.0 / math.sqrt(_DH))
    row = iq * q.shape[0] + lax.broadcasted_iota(jnp.int32, s.shape, 0)
    col = lax.broadcasted_iota(jnp.int32, s.shape, 1)
    s = jnp.where(col <= row, s, -1e30)
    m = jnp.max(s, axis=1, keepdims=True)
    e = jnp.exp(s - m)
    p = e / jnp.sum(e, axis=1, keepdims=True)
    o_ref[0] = lax.dot_general(p, v, (((1,), (0,)), ((), ())),
                               preferred_element_type=jnp.float32)


def _attn_call(q, k, v, cos, sin):
    nq = _T // _BT
    return pl.pallas_call(
        _attn_body,
        grid=(_H, nq),
        in_specs=[
            pl.BlockSpec((1, _BT, _DH), lambda h, i: (h, i, 0)),
            pl.BlockSpec((1, _T, _DH), lambda h, i: (h, 0, 0)),
            pl.BlockSpec((1, _T, _DH), lambda h, i: (h, 0, 0)),
            pl.BlockSpec((_BT, _DH), lambda h, i: (i, 0)),
            pl.BlockSpec((_BT, _DH), lambda h, i: (i, 0)),
            pl.BlockSpec((_T, _DH), lambda h, i: (0, 0)),
            pl.BlockSpec((_T, _DH), lambda h, i: (0, 0)),
        ],
        out_specs=pl.BlockSpec((1, _BT, _DH), lambda h, i: (h, i, 0)),
        out_shape=jax.ShapeDtypeStruct((_H, _T, _DH), jnp.float32),
    )(q, k, v, cos, sin, cos, sin)


def _post_body(a_ref, x_ref, aow_ref, n2_ref, rw_ref,
               x1_ref, h_ref, p_ref):
    a = lax.dot_general(a_ref[...], aow_ref[...], (((1,), (1,)), ((), ())),
                        preferred_element_type=jnp.float32)
    x1 = x_ref[...] + a
    x1_ref[...] = x1
    h = _rms(x1, n2_ref[...])
    h_ref[...] = h
    logits = lax.dot_general(h, rw_ref[...], (((1,), (1,)), ((), ())),
                             preferred_element_type=jnp.float32)
    m = jnp.max(logits, axis=1, keepdims=True)
    e = jnp.exp(logits - m)
    p_ref[...] = e / jnp.sum(e, axis=1, keepdims=True)


def _post_call(attn_merged, x2d, ao_w, n2w, router_w):
    nt = _T // _BT
    return pl.pallas_call(
        _post_body,
        grid=(nt,),
        in_specs=[
            pl.BlockSpec((_BT, _D), lambda t: (t, 0)),
            pl.BlockSpec((_BT, _D), lambda t: (t, 0)),
            pl.BlockSpec((_D, _D), lambda t: (0, 0)),
            pl.BlockSpec((1, _D), lambda t: (0, 0)),
            pl.BlockSpec((_E, _D), lambda t: (0, 0)),
        ],
        out_specs=[
            pl.BlockSpec((_BT, _D), lambda t: (t, 0)),
            pl.BlockSpec((_BT, _D), lambda t: (t, 0)),
            pl.BlockSpec((_BT, _E), lambda t: (t, 0)),
        ],
        out_shape=[
            jax.ShapeDtypeStruct((_T, _D), jnp.float32),
            jax.ShapeDtypeStruct((_T, _D), jnp.float32),
            jax.ShapeDtypeStruct((_T, _E), jnp.float32),
        ],
    )(attn_merged, x2d, ao_w, n2w, router_w)


def _make_sc_gather(n_rows, n_table, d):
    nw = 32
    per_w = n_rows // nw
    ch = 16
    n_ch = per_w // ch
    mesh = plsc.VectorSubcoreMesh(core_axis_name="c", subcore_axis_name="s")

    @functools.partial(
        pl.kernel, mesh=mesh,
        out_type=jax.ShapeDtypeStruct((n_rows, d), jnp.float32),
        scratch_types=[
            pltpu.VMEM((per_w,), jnp.int32),
            pltpu.VMEM((ch, d), jnp.float32),
            pltpu.VMEM((ch, d), jnp.float32),
            pltpu.SemaphoreType.DMA,
            pltpu.SemaphoreType.DMA,
            pltpu.SemaphoreType.DMA,
            pltpu.SemaphoreType.DMA,
        ],
    )
    def gather(table_hbm, idx_hbm, out_hbm, idx_v, buf0, buf1,
               gs0, gs1, ss0, ss1):
        wid = lax.axis_index("s") * 2 + lax.axis_index("c")
        base = wid * per_w
        pltpu.sync_copy(idx_hbm.at[pl.ds(base, per_w)], idx_v)
        bufs = (buf0, buf1)
        gsems = (gs0, gs1)
        ssems = (ss0, ss1)

        def fire(c):
            b = c % 2
            return pltpu.async_copy(
                table_hbm.at[idx_v.at[pl.ds(c * ch, ch)]], bufs[b], gsems[b])

        store_cp = [None, None]
        gcp = fire(0)
        for c in range(n_ch):
            b = c % 2
            nxt = None
            if c + 1 < n_ch:
                b2 = (c + 1) % 2
                if store_cp[b2] is not None:
                    store_cp[b2].wait()
                nxt = fire(c + 1)
            gcp.wait()
            store_cp[b] = pltpu.async_copy(
                bufs[b], out_hbm.at[pl.ds(base + c * ch, ch)], ssems[b])
            gcp = nxt
        for b in range(2):
            if store_cp[b] is not None:
                store_cp[b].wait()

    return gather


def _moe1_body(eid_ref, xs_ref, w1_ref, act_ref):
    h = lax.dot_general(xs_ref[...], w1_ref[0], (((1,), (1,)), ((), ())),
                        preferred_element_type=jnp.float32)
    act_ref[...] = h * jax.nn.sigmoid(h)


def _moe1_call(eid, xs, w1):
    return pl.pallas_call(
        _moe1_body,
        grid_spec=pltpu.PrefetchScalarGridSpec(
            num_scalar_prefetch=1,
            grid=(_NF, _NM),
            in_specs=[
                pl.BlockSpec((_BN, _D), lambda f, m, eid: (m, 0)),
                pl.BlockSpec((1, _BF, _D), lambda f, m, eid: (eid[m], f, 0)),
            ],
            out_specs=pl.BlockSpec((_BN, _BF), lambda f, m, eid: (m, f)),
        ),
        out_shape=jax.ShapeDtypeStruct((_NPAD, _DFF), jnp.float32),
    )(eid, xs, w1)


def _moe2a_body(eid_ref, act_ref, w2_ref, eo_ref):
    eo_ref[...] = lax.dot_general(
        act_ref[...], w2_ref[0], (((1,), (1,)), ((), ())),
        preferred_element_type=jnp.float32)


def _moe2b_body(eid_ref, act_ref, w2_ref, part_ref, rw_ref, eo_ref):
    o = lax.dot_general(act_ref[...], w2_ref[0], (((1,), (1,)), ((), ())),
                        preferred_element_type=jnp.float32)
    eo_ref[...] = (part_ref[...] + o) * rw_ref[...]


def _moe2_call(eid, act, w2, rw):
    part = pl.pallas_call(
        _moe2a_body,
        grid_spec=pltpu.PrefetchScalarGridSpec(
            num_scalar_prefetch=1,
            grid=(_NM,),
            in_specs=[
                pl.BlockSpec((_BN, _BF), lambda m, eid: (m, 0)),
                pl.BlockSpec((1, _D, _BF), lambda m, eid: (eid[m], 0, 0)),
            ],
            out_specs=pl.BlockSpec((_BN, _D), lambda m, eid: (m, 0)),
        ),
        out_shape=jax.ShapeDtypeStruct((_NPAD, _D), jnp.float32),
    )(eid, act, w2)
    return pl.pallas_call(
        _moe2b_body,
        grid_spec=pltpu.PrefetchScalarGridSpec(
            num_scalar_prefetch=1,
            grid=(_NM,),
            in_specs=[
                pl.BlockSpec((_BN, _BF), lambda m, eid: (m, 1)),
                pl.BlockSpec((1, _D, _BF), lambda m, eid: (eid[m], 0, 1)),
                pl.BlockSpec((_BN, _D), lambda m, eid: (m, 0)),
                pl.BlockSpec((_BN, 1), lambda m, eid: (m, 0)),
            ],
            out_specs=pl.BlockSpec((_BN, _D), lambda m, eid: (m, 0)),
        ),
        out_shape=jax.ShapeDtypeStruct((_NPAD, _D), jnp.float32),
    )(eid, act, w2, part, rw)


def _tou_body(x1_ref, g0_ref, g1_ref, n3_ref, prim_ref, tq_ref, tk_ref,
              tv_ref, to_ref, tg_ref, tgb_ref, y_ref):
    x2 = x1_ref[...] + g0_ref[...] + g1_ref[...]
    xn = _rms(x2, n3_ref[...])
    q = lax.dot_general(xn, tq_ref[...], (((1,), (1,)), ((), ())),
                        preferred_element_type=jnp.float32)
    k = lax.dot_general(prim_ref[...], tk_ref[...], (((1,), (1,)), ((), ())),
                        preferred_element_type=jnp.float32)
    v = lax.dot_general(prim_ref[...], tv_ref[...], (((1,), (1,)), ((), ())),
                        preferred_element_type=jnp.float32)
    s = lax.dot_general(q, k, (((1,), (1,)), ((), ())),
                        preferred_element_type=jnp.float32)
    s = s * (1.0 / math.sqrt(_DP))
    m = jnp.max(s, axis=1, keepdims=True)
    e = jnp.exp(s - m)
    p = e / jnp.sum(e, axis=1, keepdims=True)
    av = lax.dot_general(p, v, (((1,), (0,)), ((), ())),
                         preferred_element_type=jnp.float32)
    out = lax.dot_general(av, to_ref[...], (((1,), (1,)), ((), ())),
                          preferred_element_type=jnp.float32)
    gate_lin = jnp.sum(xn * tg_ref[...], axis=1, keepdims=True)
    gate = jax.nn.sigmoid(gate_lin + tgb_ref[0, 0])
    y_ref[...] = x2 + gate * out


def _tou_call(x1, g0, g1, n3w, prim, tq_w, tk_w, tv_w, to_w, tg_w, tg_b):
    nt = _T // _BT
    return pl.pallas_call(
        _tou_body,
        grid=(nt,),
        in_specs=[
            pl.BlockSpec((_BT, _D), lambda t: (t, 0)),
            pl.BlockSpec((_BT, _D), lambda t: (t, 0)),
            pl.BlockSpec((_BT, _D), lambda t: (t, 0)),
            pl.BlockSpec((1, _D), lambda t: (0, 0)),
            pl.BlockSpec((_NPRIM, _DP), lambda t: (0, 0)),
            pl.BlockSpec((_DP, _D), lambda t: (0, 0)),
            pl.BlockSpec((_DP, _DP), lambda t: (0, 0)),
            pl.BlockSpec((_DP, _DP), lambda t: (0, 0)),
            pl.BlockSpec((_D, _DP), lambda t: (0, 0)),
            pl.BlockSpec((1, _D), lambda t: (0, 0)),
            pl.BlockSpec((1, 1), lambda t: (0, 0)),
        ],
        out_specs=pl.BlockSpec((_BT, _D), lambda t: (t, 0)),
        out_shape=jax.ShapeDtypeStruct((_T, _D), jnp.float32),
    )(x1, g0, g1, n3w, prim, tq_w, tk_w, tv_w, to_w, tg_w, tg_b)


def kernel(x, tou_embeds, norm1_w, qkv_w, ao_w, norm2_w, router_w, w1, w2,
           norm3_w, tq_w, tk_w, tv_w, to_w, tg_w, tg_b):
    x2d = x[0]
    n1w = norm1_w.reshape(1, _D)
    n2w = norm2_w.reshape(1, _D)
    n3w = norm3_w.reshape(1, _D)
    qkv_w3 = qkv_w.reshape(3, _D, _D)

    inv_freq = 1.0 / _ROPE_THETA ** (
        jnp.arange(0, _DH, 2, dtype=jnp.float32) / _DH)
    freqs = jnp.outer(jnp.arange(_T, dtype=jnp.float32), inv_freq)
    emb = jnp.concatenate([freqs, freqs], axis=-1)
    cos = jnp.cos(emb)
    sin = jnp.sin(emb)

    qkv = _qkv_call(x2d, qkv_w3, n1w)
    qh = qkv[0].reshape(_T, _H, _DH).transpose(1, 0, 2)
    kh = qkv[1].reshape(_T, _H, _DH).transpose(1, 0, 2)
    vh = qkv[2].reshape(_T, _H, _DH).transpose(1, 0, 2)
    attn = _attn_call(qh, kh, vh, cos, sin)
    attn_merged = attn.transpose(1, 0, 2).reshape(_T, _D)
    x1, h, probs = _post_call(attn_merged, x2d, ao_w, n2w, router_w)

    topv, topi = lax.top_k(probs, _TOPK)
    topv = topv / jnp.sum(topv, axis=-1, keepdims=True)
    a_flat = topi.reshape(-1).astype(jnp.int32)
    oh = (a_flat[:, None] == jnp.arange(_E)[None, :]).astype(jnp.int32)
    ranks = jnp.cumsum(oh, axis=0) - oh
    rank_flat = jnp.sum(ranks * oh, axis=1)
    g = jnp.sum(oh, axis=0)
    gp = ((g + _BN - 1) // _BN) * _BN
    o_end = jnp.cumsum(gp)
    o_start = o_end - gp
    pos_flat = o_start[a_flat] + rank_flat
    tok_of = jnp.arange(_T * _TOPK, dtype=jnp.int32) // _TOPK
    src_tok = jnp.zeros((_NPAD,), jnp.int32).at[pos_flat].set(tok_of)
    rw = jnp.zeros((_NPAD, 1), jnp.float32).at[pos_flat, 0].set(
        topv.reshape(-1))
    tile_start = jnp.arange(_NM) * _BN
    eid = jnp.clip(jnp.searchsorted(o_end, tile_start, side="right"),
                   0, _E - 1).astype(jnp.int32)

    f = g.astype(jnp.float32) / _T
    pm = jnp.mean(probs, axis=0)
    aux = _E * jnp.sum(f * pm)

    xs = _make_sc_gather(_NPAD, _T, _D)(h, src_tok)

    act = _moe1_call(eid, xs, w1)
    eo = _moe2_call(eid, act, w2, rw)

    pos_km = pos_flat.reshape(_T, _TOPK).T.reshape(-1)
    gout = _make_sc_gather(_T * _TOPK, _NPAD, _D)(eo, pos_km)
    g0 = gout[:_T]
    g1 = gout[_T:]

    y = _tou_call(x1, g0, g1, n3w, tou_embeds, tq_w, tk_w, tv_w, to_w,
                  tg_w, tg_b.reshape(1, 1))
    return (y.reshape(1, _T, _D), aux)
